# Initial kernel scaffold; baseline (speedup 1.0000x reference)
import jax
import jax.numpy as jnp
from jax.experimental import pallas as pl


def kernel(query_points, positions, log_sigma, K_neighbors):
    raise NotImplementedError("write your pallas kernel here")


# speedup vs baseline: 12.1744x; 12.1744x over previous
import functools

import jax
import jax.numpy as jnp
from jax import lax
from jax.experimental import pallas as pl
from jax.experimental.pallas import tpu as pltpu
from jax.experimental.pallas import tpu_sc as plsc

L = 16
K_OUT = 10


def _make_sc_kernel(n_queries: int, n_pts: int):
    info = plsc.get_sparse_core_info()
    nc, ns = info.num_cores, info.num_subcores
    nw = nc * ns
    assert n_queries % nw == 0
    qw = n_queries // nw
    nchunk = n_pts // L
    owords = qw * K_OUT

    mesh = plsc.VectorSubcoreMesh(core_axis_name="c", subcore_axis_name="s")

    @functools.partial(
        pl.kernel,
        out_type=(
            jax.ShapeDtypeStruct((n_queries * K_OUT,), jnp.float32),
            jax.ShapeDtypeStruct((n_queries * K_OUT,), jnp.int32),
        ),
        mesh=mesh,
        compiler_params=pltpu.CompilerParams(needs_layout_passes=False),
        scratch_types=[
            pltpu.VMEM((n_pts,), jnp.float32),
            pltpu.VMEM((n_pts,), jnp.float32),
            pltpu.VMEM((n_pts,), jnp.float32),
            pltpu.VMEM((n_pts,), jnp.float32),
            pltpu.VMEM((n_pts,), jnp.float32),
            pltpu.VMEM((qw * 3 + L,), jnp.float32),
            pltpu.VMEM((n_pts,), jnp.float32),
            pltpu.VMEM((n_pts + L,), jnp.float32),
            pltpu.VMEM((n_pts + L,), jnp.int32),
            pltpu.VMEM((owords + L,), jnp.float32),
            pltpu.VMEM((owords + L,), jnp.int32),
        ],
    )
    def sc_kernel(q_hbm, px_hbm, py_hbm, pz_hbm, p2_hbm, s2_hbm,
                  outw_hbm, outi_hbm,
                  pxv, pyv, pzv, p2v, s2v, qv, dbuf, candk, candi,
                  wbuf, ibuf):
        wid = lax.axis_index("s") * nc + lax.axis_index("c")
        pltpu.sync_copy(px_hbm, pxv)
        pltpu.sync_copy(py_hbm, pyv)
        pltpu.sync_copy(pz_hbm, pzv)
        pltpu.sync_copy(p2_hbm, p2v)
        pltpu.sync_copy(s2_hbm, s2v)
        pltpu.sync_copy(q_hbm.at[pl.ds(wid * (qw * 3), qw * 3)],
                        qv.at[pl.ds(0, qw * 3)])

        inf16 = jnp.full((L,), jnp.inf, jnp.float32)
        lanes = lax.iota(jnp.int32, L)

        def per_query(i, carry):
            qc = qv[pl.ds(3 * i, L)]
            qx = qc[0]
            qy = qc[1]
            qz = qc[2]
            q2 = qx * qx + qy * qy + qz * qz
            qbits = plsc.bitcast(qc, jnp.int32)
            rnd = ((qbits >> 16) & 1) + 0x7FFF
            qcb = plsc.bitcast((qbits + rnd) & jnp.int32(-65536), jnp.float32)
            ax = -2.0 * qcb[0]
            ay = -2.0 * qcb[1]
            az = -2.0 * qcb[2]

            def pass_a(j, m):
                d2 = jnp.maximum(
                    (p2v[pl.ds(L * j, L)] + q2)
                    + (ax * pxv[pl.ds(L * j, L)]
                       + ay * pyv[pl.ds(L * j, L)]
                       + az * pzv[pl.ds(L * j, L)]),
                    0.0,
                )
                dbuf[pl.ds(L * j, L)] = d2
                return jnp.minimum(m, d2)

            m = lax.fori_loop(0, nchunk, pass_a, inf16, unroll=4)
            pivot = jnp.max(m)

            def pass_b(j, cnt):
                d2 = dbuf[pl.ds(L * j, L)]
                msk = d2 <= pivot
                idx = lanes + L * j
                plsc.store_compressed(candk.at[pl.ds(cnt, L)], d2, mask=msk)
                plsc.store_compressed(candi.at[pl.ds(cnt, L)], idx, mask=msk)
                n = plsc.all_reduce_population_count(msk)
                return cnt + n[0]

            cnt = lax.fori_loop(0, nchunk, pass_b, jnp.int32(0), unroll=4)
            candk[pl.ds(cnt, L)] = inf16
            candi[pl.ds(cnt, L)] = lanes

            tk, tv = plsc.sort_key_val(candk[pl.ds(0, L)], candi[pl.ds(0, L)])
            nch = (cnt + (L - 1)) // L

            def merge(j, c):
                tk, tv = c
                sk, sv = plsc.sort_key_val(candk[pl.ds(L * j, L)],
                                           candi[pl.ds(L * j, L)])
                rk = lax.rev(sk, (0,))
                rv = lax.rev(sv, (0,))
                sel = tk <= rk
                mk = jnp.minimum(tk, rk)
                mv = jnp.where(sel, tv, rv)
                mk, mv = plsc.sort_key_val(mk, mv)
                return (mk, mv)

            tk, tv = lax.fori_loop(1, nch, merge, (tk, tv))

            s2g = plsc.load_gather(s2v, [tv])
            beta = jnp.exp(-0.5 * tk / s2g)
            beta = jnp.where(lanes < K_OUT, beta, 0.0)
            w = beta / (jnp.sum(beta) + 1e-8)
            wbuf[pl.ds(K_OUT * i, L)] = w
            ibuf[pl.ds(K_OUT * i, L)] = tv
            return carry

        lax.fori_loop(0, qw, per_query, jnp.int32(0))

        pltpu.sync_copy(wbuf.at[pl.ds(0, owords)],
                        outw_hbm.at[pl.ds(wid * owords, owords)])
        pltpu.sync_copy(ibuf.at[pl.ds(0, owords)],
                        outi_hbm.at[pl.ds(wid * owords, owords)])

    return sc_kernel


def kernel(query_points, positions, log_sigma, K_neighbors):
    n_queries = query_points.shape[0]
    n_pts = positions.shape[0]
    pbits = lax.bitcast_convert_type(positions, jnp.int32)
    prnd = ((pbits >> 16) & 1) + 0x7FFF
    posb = lax.bitcast_convert_type((pbits + prnd) & jnp.int32(-65536),
                                    jnp.float32)
    px = posb[:, 0]
    py = posb[:, 1]
    pz = posb[:, 2]
    p2 = jnp.sum(positions * positions, axis=1)
    sigma = jnp.exp(log_sigma)
    s2 = sigma * sigma + 1e-8
    qflat = query_points.reshape(-1)

    wflat, iflat = _make_sc_kernel(n_queries, n_pts)(qflat, px, py, pz, p2, s2)
    return (wflat.reshape(n_queries, K_OUT), iflat.reshape(n_queries, K_OUT))


# speedup vs baseline: 16.3453x; 1.3426x over previous
import functools

import jax
import jax.numpy as jnp
from jax import lax
from jax.experimental import pallas as pl
from jax.experimental.pallas import tpu as pltpu
from jax.experimental.pallas import tpu_sc as plsc

L = 16
K_OUT = 10


def _make_sc_kernel(n_queries: int, n_pts: int):
    info = plsc.get_sparse_core_info()
    nc, ns = info.num_cores, info.num_subcores
    nw = nc * ns
    assert n_queries % (2 * nw) == 0
    qw = n_queries // nw
    nchunk = n_pts // L
    owords = qw * K_OUT

    mesh = plsc.VectorSubcoreMesh(core_axis_name="c", subcore_axis_name="s")

    @functools.partial(
        pl.kernel,
        out_type=(
            jax.ShapeDtypeStruct((n_queries * K_OUT,), jnp.float32),
            jax.ShapeDtypeStruct((n_queries * K_OUT,), jnp.int32),
        ),
        mesh=mesh,
        compiler_params=pltpu.CompilerParams(needs_layout_passes=False),
        scratch_types=[
            pltpu.VMEM((n_pts,), jnp.float32),
            pltpu.VMEM((n_pts,), jnp.float32),
            pltpu.VMEM((n_pts,), jnp.float32),
            pltpu.VMEM((n_pts,), jnp.float32),
            pltpu.VMEM((n_pts,), jnp.float32),
            pltpu.VMEM((qw * 3 + L,), jnp.float32),
            pltpu.VMEM((n_pts,), jnp.float32),
            pltpu.VMEM((n_pts,), jnp.float32),
            pltpu.VMEM((n_pts + L,), jnp.float32),
            pltpu.VMEM((n_pts + L,), jnp.int32),
            pltpu.VMEM((n_pts + L,), jnp.float32),
            pltpu.VMEM((n_pts + L,), jnp.int32),
            pltpu.VMEM((owords + L,), jnp.float32),
            pltpu.VMEM((owords + L,), jnp.int32),
        ],
    )
    def sc_kernel(q_hbm, px_hbm, py_hbm, pz_hbm, p2_hbm, s2_hbm,
                  outw_hbm, outi_hbm,
                  pxv, pyv, pzv, p2v, s2v, qv, dbuf0, dbuf1,
                  candk0, candi0, candk1, candi1, wbuf, ibuf):
        wid = lax.axis_index("s") * nc + lax.axis_index("c")
        pltpu.sync_copy(px_hbm, pxv)
        pltpu.sync_copy(py_hbm, pyv)
        pltpu.sync_copy(pz_hbm, pzv)
        pltpu.sync_copy(p2_hbm, p2v)
        pltpu.sync_copy(s2_hbm, s2v)
        pltpu.sync_copy(q_hbm.at[pl.ds(wid * (qw * 3), qw * 3)],
                        qv.at[pl.ds(0, qw * 3)])

        inf16 = jnp.full((L,), jnp.inf, jnp.float32)
        lanes = lax.iota(jnp.int32, L)

        def topk_weights(i, candk, candi, cnt):
            candk[pl.ds(cnt, L)] = inf16
            candi[pl.ds(cnt, L)] = lanes
            tk, tv = plsc.sort_key_val(candk[pl.ds(0, L)], candi[pl.ds(0, L)])
            nch = (cnt + (L - 1)) // L

            def merge(j, c):
                tk, tv = c
                sk, sv = plsc.sort_key_val(candk[pl.ds(L * j, L)],
                                           candi[pl.ds(L * j, L)])
                rk = lax.rev(sk, (0,))
                rv = lax.rev(sv, (0,))
                sel = tk <= rk
                mk = jnp.minimum(tk, rk)
                mv = jnp.where(sel, tv, rv)
                mk, mv = plsc.sort_key_val(mk, mv)
                return (mk, mv)

            tk, tv = lax.fori_loop(1, nch, merge, (tk, tv))

            s2g = plsc.load_gather(s2v, [tv])
            beta = jnp.exp(-0.5 * jnp.maximum(tk, 0.0) / s2g)
            beta = jnp.where(lanes < K_OUT, beta, 0.0)
            w = beta / (jnp.sum(beta) + 1e-8)
            wbuf[pl.ds(K_OUT * i, L)] = w
            ibuf[pl.ds(K_OUT * i, L)] = tv

        def per_pair(ip, carry):
            i0 = 2 * ip
            qc = qv[pl.ds(3 * i0, L)]
            qbits = plsc.bitcast(qc, jnp.int32)
            rnd = ((qbits >> 16) & 1) + 0x7FFF
            qcb = plsc.bitcast((qbits + rnd) & jnp.int32(-65536), jnp.float32)
            q2_0 = qc[0] * qc[0] + qc[1] * qc[1] + qc[2] * qc[2]
            q2_1 = qc[3] * qc[3] + qc[4] * qc[4] + qc[5] * qc[5]
            ax0 = -2.0 * qcb[0]
            ay0 = -2.0 * qcb[1]
            az0 = -2.0 * qcb[2]
            ax1 = -2.0 * qcb[3]
            ay1 = -2.0 * qcb[4]
            az1 = -2.0 * qcb[5]

            def pass_a(j, c):
                m0, m1 = c
                pxc = pxv[pl.ds(L * j, L)]
                pyc = pyv[pl.ds(L * j, L)]
                pzc = pzv[pl.ds(L * j, L)]
                p2c = p2v[pl.ds(L * j, L)]
                d2_0 = (p2c + q2_0) + (ax0 * pxc + ay0 * pyc + az0 * pzc)
                d2_1 = (p2c + q2_1) + (ax1 * pxc + ay1 * pyc + az1 * pzc)
                dbuf0[pl.ds(L * j, L)] = d2_0
                dbuf1[pl.ds(L * j, L)] = d2_1
                return (jnp.minimum(m0, d2_0), jnp.minimum(m1, d2_1))

            m0, m1 = lax.fori_loop(0, nchunk, pass_a, (inf16, inf16),
                                   unroll=4)
            pivot0 = lax.sort(m0)[K_OUT - 1]
            pivot1 = lax.sort(m1)[K_OUT - 1]

            def pass_b(j, c):
                cnt0, cnt1 = c
                idx = lanes + L * j
                d2_0 = dbuf0[pl.ds(L * j, L)]
                d2_1 = dbuf1[pl.ds(L * j, L)]
                msk0 = d2_0 <= pivot0
                msk1 = d2_1 <= pivot1
                n0 = plsc.all_reduce_population_count(msk0)[0]
                n1 = plsc.all_reduce_population_count(msk1)[0]

                @pl.when(n0 > 0)
                def _():
                    plsc.store_compressed(candk0.at[pl.ds(cnt0, L)], d2_0,
                                          mask=msk0)
                    plsc.store_compressed(candi0.at[pl.ds(cnt0, L)], idx,
                                          mask=msk0)

                @pl.when(n1 > 0)
                def _():
                    plsc.store_compressed(candk1.at[pl.ds(cnt1, L)], d2_1,
                                          mask=msk1)
                    plsc.store_compressed(candi1.at[pl.ds(cnt1, L)], idx,
                                          mask=msk1)

                return (cnt0 + n0, cnt1 + n1)

            cnt0, cnt1 = lax.fori_loop(0, nchunk, pass_b,
                                       (jnp.int32(0), jnp.int32(0)),
                                       unroll=4)

            topk_weights(i0, candk0, candi0, cnt0)
            topk_weights(i0 + 1, candk1, candi1, cnt1)
            return carry

        lax.fori_loop(0, qw // 2, per_pair, jnp.int32(0))

        pltpu.sync_copy(wbuf.at[pl.ds(0, owords)],
                        outw_hbm.at[pl.ds(wid * owords, owords)])
        pltpu.sync_copy(ibuf.at[pl.ds(0, owords)],
                        outi_hbm.at[pl.ds(wid * owords, owords)])

    return sc_kernel


def kernel(query_points, positions, log_sigma, K_neighbors):
    n_queries = query_points.shape[0]
    n_pts = positions.shape[0]
    pbits = lax.bitcast_convert_type(positions, jnp.int32)
    prnd = ((pbits >> 16) & 1) + 0x7FFF
    posb = lax.bitcast_convert_type((pbits + prnd) & jnp.int32(-65536),
                                    jnp.float32)
    px = posb[:, 0]
    py = posb[:, 1]
    pz = posb[:, 2]
    p2 = jnp.sum(positions * positions, axis=1)
    sigma = jnp.exp(log_sigma)
    s2 = sigma * sigma + 1e-8
    qflat = query_points.reshape(-1)

    wflat, iflat = _make_sc_kernel(n_queries, n_pts)(qflat, px, py, pz, p2, s2)
    return (wflat.reshape(n_queries, K_OUT), iflat.reshape(n_queries, K_OUT))


# speedup vs baseline: 47.0423x; 2.8780x over previous
import functools

import jax
import jax.numpy as jnp
from jax import lax
from jax.experimental import pallas as pl
from jax.experimental.pallas import tpu as pltpu
from jax.experimental.pallas import tpu_sc as plsc

L = 16
K_OUT = 10


def _make_sc_kernel(n_queries: int, n_pts: int):
    info = plsc.get_sparse_core_info()
    nc, ns = info.num_cores, info.num_subcores
    nw = nc * ns
    assert n_queries % (2 * nw) == 0
    qw = n_queries // nw
    nchunk = n_pts // L
    owords = qw * K_OUT

    mesh = plsc.VectorSubcoreMesh(core_axis_name="c", subcore_axis_name="s")

    @functools.partial(
        pl.kernel,
        out_type=(
            jax.ShapeDtypeStruct((n_queries * K_OUT,), jnp.float32),
            jax.ShapeDtypeStruct((n_queries * K_OUT,), jnp.int32),
        ),
        mesh=mesh,
        compiler_params=pltpu.CompilerParams(needs_layout_passes=False),
        scratch_types=[
            pltpu.VMEM((n_pts,), jnp.float32),
            pltpu.VMEM((n_pts,), jnp.float32),
            pltpu.VMEM((n_pts,), jnp.float32),
            pltpu.VMEM((n_pts,), jnp.float32),
            pltpu.VMEM((n_pts,), jnp.float32),
            pltpu.VMEM((qw * 3 + L,), jnp.float32),
            pltpu.VMEM((n_pts,), jnp.float32),
            pltpu.VMEM((n_pts,), jnp.float32),
            pltpu.VMEM((n_pts + L,), jnp.float32),
            pltpu.VMEM((n_pts + L,), jnp.int32),
            pltpu.VMEM((n_pts + L,), jnp.float32),
            pltpu.VMEM((n_pts + L,), jnp.int32),
            pltpu.VMEM((owords + L,), jnp.float32),
            pltpu.VMEM((owords + L,), jnp.int32),
        ],
    )
    def sc_kernel(q_hbm, px_hbm, py_hbm, pz_hbm, p2_hbm, s2_hbm,
                  outw_hbm, outi_hbm,
                  pxv, pyv, pzv, p2v, s2v, qv, dbuf0, dbuf1,
                  candk0, candi0, candk1, candi1, wbuf, ibuf):
        wid = lax.axis_index("s") * nc + lax.axis_index("c")
        pltpu.sync_copy(px_hbm, pxv)
        pltpu.sync_copy(py_hbm, pyv)
        pltpu.sync_copy(pz_hbm, pzv)
        pltpu.sync_copy(p2_hbm, p2v)
        pltpu.sync_copy(s2_hbm, s2v)
        pltpu.sync_copy(q_hbm.at[pl.ds(wid * (qw * 3), qw * 3)],
                        qv.at[pl.ds(0, qw * 3)])

        inf16 = jnp.full((L,), jnp.inf, jnp.float32)
        lanes = lax.iota(jnp.int32, L)

        def topk_weights(i, candk, candi, cnt):
            candk[pl.ds(cnt, L)] = inf16
            candi[pl.ds(cnt, L)] = lanes
            tk, tv = plsc.sort_key_val(candk[pl.ds(0, L)], candi[pl.ds(0, L)])
            nch = (cnt + (L - 1)) // L

            def merge(j, c):
                tk, tv = c
                sk, sv = plsc.sort_key_val(candk[pl.ds(L * j, L)],
                                           candi[pl.ds(L * j, L)])
                rk = lax.rev(sk, (0,))
                rv = lax.rev(sv, (0,))
                sel = tk <= rk
                mk = jnp.minimum(tk, rk)
                mv = jnp.where(sel, tv, rv)
                mk, mv = plsc.sort_key_val(mk, mv)
                return (mk, mv)

            tk, tv = lax.fori_loop(1, nch, merge, (tk, tv))

            s2g = plsc.load_gather(s2v, [tv])
            beta = jnp.exp(-0.5 * jnp.maximum(tk, 0.0) / s2g)
            beta = jnp.where(lanes < K_OUT, beta, 0.0)
            w = beta / (jnp.sum(beta) + 1e-8)
            wbuf[pl.ds(K_OUT * i, L)] = w
            ibuf[pl.ds(K_OUT * i, L)] = tv

        def per_pair(ip, carry):
            i0 = 2 * ip
            qc = qv[pl.ds(3 * i0, L)]
            qbits = plsc.bitcast(qc, jnp.int32)
            rnd = ((qbits >> 16) & 1) + 0x7FFF
            qcb = plsc.bitcast((qbits + rnd) & jnp.int32(-65536), jnp.float32)
            q2_0 = qc[0] * qc[0] + qc[1] * qc[1] + qc[2] * qc[2]
            q2_1 = qc[3] * qc[3] + qc[4] * qc[4] + qc[5] * qc[5]
            ax0 = -2.0 * qcb[0]
            ay0 = -2.0 * qcb[1]
            az0 = -2.0 * qcb[2]
            ax1 = -2.0 * qcb[3]
            ay1 = -2.0 * qcb[4]
            az1 = -2.0 * qcb[5]

            @plsc.parallel_loop(0, nchunk, unroll=4, carry=(inf16, inf16))
            def pass_a(j, c):
                m0, m1 = c
                pxc = pxv[pl.ds(L * j, L)]
                pyc = pyv[pl.ds(L * j, L)]
                pzc = pzv[pl.ds(L * j, L)]
                p2c = p2v[pl.ds(L * j, L)]
                d2_0 = (p2c + q2_0) + (ax0 * pxc + ay0 * pyc + az0 * pzc)
                d2_1 = (p2c + q2_1) + (ax1 * pxc + ay1 * pyc + az1 * pzc)
                dbuf0[pl.ds(L * j, L)] = d2_0
                dbuf1[pl.ds(L * j, L)] = d2_1
                return (jnp.minimum(m0, d2_0), jnp.minimum(m1, d2_1))

            m0, m1 = pass_a
            pivot0 = lax.sort(m0)[K_OUT - 1]
            pivot1 = lax.sort(m1)[K_OUT - 1]

            @plsc.parallel_loop(0, nchunk, unroll=4,
                                carry=(jnp.int32(0), jnp.int32(0)))
            def pass_b(j, c):
                cnt0, cnt1 = c
                idx = lanes + L * j
                d2_0 = dbuf0[pl.ds(L * j, L)]
                d2_1 = dbuf1[pl.ds(L * j, L)]
                msk0 = d2_0 <= pivot0
                msk1 = d2_1 <= pivot1
                n0 = plsc.all_reduce_population_count(msk0)[0]
                n1 = plsc.all_reduce_population_count(msk1)[0]
                plsc.store_compressed(candk0.at[pl.ds(cnt0, L)], d2_0,
                                      mask=msk0)
                plsc.store_compressed(candi0.at[pl.ds(cnt0, L)], idx,
                                      mask=msk0)
                plsc.store_compressed(candk1.at[pl.ds(cnt1, L)], d2_1,
                                      mask=msk1)
                plsc.store_compressed(candi1.at[pl.ds(cnt1, L)], idx,
                                      mask=msk1)
                return (cnt0 + n0, cnt1 + n1)

            cnt0, cnt1 = pass_b

            topk_weights(i0, candk0, candi0, cnt0)
            topk_weights(i0 + 1, candk1, candi1, cnt1)
            return carry

        lax.fori_loop(0, qw // 2, per_pair, jnp.int32(0))

        pltpu.sync_copy(wbuf.at[pl.ds(0, owords)],
                        outw_hbm.at[pl.ds(wid * owords, owords)])
        pltpu.sync_copy(ibuf.at[pl.ds(0, owords)],
                        outi_hbm.at[pl.ds(wid * owords, owords)])

    return sc_kernel


def kernel(query_points, positions, log_sigma, K_neighbors):
    n_queries = query_points.shape[0]
    n_pts = positions.shape[0]
    pbits = lax.bitcast_convert_type(positions, jnp.int32)
    prnd = ((pbits >> 16) & 1) + 0x7FFF
    posb = lax.bitcast_convert_type((pbits + prnd) & jnp.int32(-65536),
                                    jnp.float32)
    px = posb[:, 0]
    py = posb[:, 1]
    pz = posb[:, 2]
    p2 = jnp.sum(positions * positions, axis=1)
    sigma = jnp.exp(log_sigma)
    s2 = sigma * sigma + 1e-8
    qflat = query_points.reshape(-1)

    wflat, iflat = _make_sc_kernel(n_queries, n_pts)(qflat, px, py, pz, p2, s2)
    return (wflat.reshape(n_queries, K_OUT), iflat.reshape(n_queries, K_OUT))
